# baseline (speedup 1.0000x reference)
import functools

import jax
import jax.numpy as jnp
from jax import lax
from jax.experimental import pallas as pl
from jax.experimental.pallas import tpu as pltpu
from jax.experimental.pallas import tpu_sc as plsc

N = 10000
D = 128
NC = 2
NS = 16
NW = NC * NS
K = 128
N_ACC = 10112
RPT = N_ACC // NS

_mesh = plsc.VectorSubcoreMesh(core_axis_name="c", subcore_axis_name="s")


def _cdiv(a, b):
    return (a + b - 1) // b


def _make_deg_kernel(cpt):
    ept = cpt * K

    @functools.partial(
        pl.kernel,
        out_type=jax.ShapeDtypeStruct((NW, 1, N_ACC), jnp.float32),
        mesh=_mesh,
        scratch_types=[
            pltpu.VMEM((ept,), jnp.int32),
            pltpu.VMEM((N_ACC,), jnp.float32),
        ],
        compiler_params=pltpu.CompilerParams(needs_layout_passes=False),
    )
    def deg_kernel(dst_hbm, out_hbm, dstb, hist):
        c = lax.axis_index("c")
        s = lax.axis_index("s")
        wid = c * NS + s
        pltpu.sync_copy(dst_hbm.at[pl.ds(wid * ept, ept)], dstb)

        def zbody(i, carry):
            hist[pl.ds(i * 16, 16)] = jnp.zeros((16,), jnp.float32)
            return carry

        lax.fori_loop(0, N_ACC // 16, zbody, 0)
        ones16 = jnp.ones((16,), jnp.float32)

        def body(i, carry):
            idx = dstb[pl.ds(i * 16, 16)]
            plsc.addupdate_scatter(hist, [idx], ones16)
            return carry

        lax.fori_loop(0, ept // 16, body, 0)
        pltpu.sync_copy(hist, out_hbm.at[wid, 0])

    return deg_kernel


def _make_agg_kernel(cpt):
    @functools.partial(
        pl.kernel,
        out_type=jax.ShapeDtypeStruct((NC, N_ACC, D), jnp.float32),
        mesh=_mesh,
        scratch_types=[
            pltpu.VMEM((K,), jnp.int32),
            pltpu.VMEM((K,), jnp.int32),
            pltpu.VMEM((K,), jnp.int32),
            pltpu.VMEM((K,), jnp.int32),
            pltpu.VMEM((K, D), jnp.float32),
            pltpu.VMEM((K, D), jnp.float32),
            pltpu.VMEM_SHARED((N_ACC, D), jnp.float32),
            pltpu.SemaphoreType.DMA,
            pltpu.SemaphoreType.DMA,
        ],
    )
    def agg_kernel(h_hbm, src_hbm, dst_hbm, zeros_hbm, out_hbm,
                   srcA, dstA, srcB, dstB, rowsA, rowsB, acc, gsem, ssem):
        c = lax.axis_index("c")
        s = lax.axis_index("s")
        wid = c * NS + s
        pltpu.sync_copy(
            zeros_hbm.at[pl.ds(s * RPT, RPT)],
            acc.at[pl.ds(s * RPT, RPT)],
        )
        plsc.subcore_barrier()

        def body(i, carry):
            base = (wid * cpt + 2 * i) * K
            pltpu.sync_copy(src_hbm.at[pl.ds(base, K)], srcA)
            pltpu.sync_copy(dst_hbm.at[pl.ds(base, K)], dstA)
            dA = pltpu.async_copy(h_hbm.at[srcA], rowsA, gsem)
            pltpu.sync_copy(src_hbm.at[pl.ds(base + K, K)], srcB)
            pltpu.sync_copy(dst_hbm.at[pl.ds(base + K, K)], dstB)
            dA.wait()
            dB = pltpu.async_copy(h_hbm.at[srcB], rowsB, gsem)
            sA = pltpu.async_copy(rowsA, acc.at[dstA], ssem, add=True)
            dB.wait()
            sA.wait()
            pltpu.sync_copy(rowsB, acc.at[dstB], add=True)
            return carry

        lax.fori_loop(0, cpt // 2, body, 0)
        plsc.subcore_barrier()
        pltpu.sync_copy(
            acc.at[pl.ds(s * RPT, RPT)],
            out_hbm.at[c, pl.ds(s * RPT, RPT)],
        )

    return agg_kernel


def _tc_first_body(h_ref, x_ref, w_ref, dinv_ref, hp_ref):
    ones_w = jnp.ones((NW, 1), jnp.float32)
    deg = 1.0 + lax.dot_general(
        h_ref[...], ones_w, (((0,), (0,)), ((), ())),
        preferred_element_type=jnp.float32)
    dinv = lax.rsqrt(deg)
    y = jnp.dot(x_ref[...], w_ref[...], preferred_element_type=jnp.float32)
    dinv_ref[...] = dinv
    hp_ref[...] = y * dinv


_tc_first = pl.pallas_call(
    _tc_first_body,
    out_shape=[
        jax.ShapeDtypeStruct((N, 1), jnp.float32),
        jax.ShapeDtypeStruct((N, D), jnp.float32),
    ],
)


def _tc_mid_body(p0_ref, p1_ref, hp_ref, dinv_ref, b_ref, g_ref, be_ref,
                 w_ref, out_ref):
    dinv = dinv_ref[...]
    t = dinv * (p0_ref[...] + p1_ref[...] + hp_ref[...]) + b_ref[...]
    m = jnp.mean(t, axis=0, keepdims=True)
    cen = t - m
    v = jnp.mean(cen * cen, axis=0, keepdims=True)
    tn = cen * lax.rsqrt(v + 1e-5) * g_ref[...] + be_ref[...]
    h = jnp.maximum(tn, 0.0)
    y = jnp.dot(h, w_ref[...], preferred_element_type=jnp.float32)
    out_ref[...] = y * dinv


_tc_mid = pl.pallas_call(
    _tc_mid_body,
    out_shape=jax.ShapeDtypeStruct((N, D), jnp.float32),
)


def _tc_final_body(p0_ref, p1_ref, hp_ref, dinv_ref, b_ref, out_ref):
    t = dinv_ref[...] * (p0_ref[...] + p1_ref[...] + hp_ref[...]) + b_ref[...]
    mx = jnp.max(t, axis=1, keepdims=True)
    ex = jnp.exp(t - mx)
    lse = jnp.log(jnp.sum(ex, axis=1, keepdims=True)) + mx
    out_ref[...] = t - lse


_tc_final = pl.pallas_call(
    _tc_final_body,
    out_shape=jax.ShapeDtypeStruct((N, D), jnp.float32),
)


def kernel(x, edge_index, W1, b1, g1, be1, W2, b2, g2, be2, Wl, bl):
    e = edge_index.shape[1]
    cpt = _cdiv(e, NW * K)
    cpt += cpt % 2
    e_pad = NW * K * cpt
    pad = e_pad - e

    src = edge_index[0].astype(jnp.int32)
    dst = edge_index[1].astype(jnp.int32)
    src_p = jnp.concatenate([src, jnp.zeros((pad,), jnp.int32)])
    dst_p = jnp.concatenate([dst, jnp.full((pad,), N, jnp.int32)])

    zeros_agg = jnp.zeros((N_ACC, D), jnp.float32)

    hists = _make_deg_kernel(cpt)(dst_p)[:, 0, :N]

    b1r = b1.reshape(1, D)
    g1r = g1.reshape(1, D)
    be1r = be1.reshape(1, D)
    b2r = b2.reshape(1, D)
    g2r = g2.reshape(1, D)
    be2r = be2.reshape(1, D)
    blr = bl.reshape(1, D)

    agg = _make_agg_kernel(cpt)

    dinv, hp1 = _tc_first(hists, x, W1)
    p1 = agg(hp1, src_p, dst_p, zeros_agg)
    hp2 = _tc_mid(p1[0, :N], p1[1, :N], hp1, dinv, b1r, g1r, be1r, W2)
    p2 = agg(hp2, src_p, dst_p, zeros_agg)
    hp3 = _tc_mid(p2[0, :N], p2[1, :N], hp2, dinv, b2r, g2r, be2r, Wl)
    p3 = agg(hp3, src_p, dst_p, zeros_agg)
    return _tc_final(p3[0, :N], p3[1, :N], hp3, dinv, blr)


# speedup vs baseline: 1.3170x; 1.3170x over previous
import functools

import jax
import jax.numpy as jnp
from jax import lax
from jax.experimental import pallas as pl
from jax.experimental.pallas import tpu as pltpu
from jax.experimental.pallas import tpu_sc as plsc

N = 10000
D = 128
NC = 2
NS = 16
NW = NC * NS
K = 128
N_ACC = 10112
RPT = N_ACC // NS

_mesh = plsc.VectorSubcoreMesh(core_axis_name="c", subcore_axis_name="s")


def _cdiv(a, b):
    return (a + b - 1) // b


def _make_deg_kernel(cpt):
    ept = cpt * K

    @functools.partial(
        pl.kernel,
        out_type=jax.ShapeDtypeStruct((NW, 1, N_ACC), jnp.float32),
        mesh=_mesh,
        scratch_types=[
            pltpu.VMEM((ept,), jnp.int32),
            pltpu.VMEM((N_ACC,), jnp.float32),
        ],
        compiler_params=pltpu.CompilerParams(needs_layout_passes=False),
    )
    def deg_kernel(dst_hbm, out_hbm, dstb, hist):
        c = lax.axis_index("c")
        s = lax.axis_index("s")
        wid = c * NS + s
        pltpu.sync_copy(dst_hbm.at[pl.ds(wid * ept, ept)], dstb)

        def zbody(i, carry):
            hist[pl.ds(i * 16, 16)] = jnp.zeros((16,), jnp.float32)
            return carry

        lax.fori_loop(0, N_ACC // 16, zbody, 0)
        ones16 = jnp.ones((16,), jnp.float32)

        def body(i, carry):
            idx = dstb[pl.ds(i * 16, 16)]
            plsc.addupdate_scatter(hist, [idx], ones16)
            return carry

        lax.fori_loop(0, ept // 16, body, 0)
        pltpu.sync_copy(hist, out_hbm.at[wid, 0])

    return deg_kernel


def _make_agg_kernel(cpt):
    @functools.partial(
        pl.kernel,
        out_type=jax.ShapeDtypeStruct((NC, N_ACC, D), jnp.float32),
        mesh=_mesh,
        scratch_types=[
            pltpu.VMEM((K,), jnp.int32),
            pltpu.VMEM((K,), jnp.int32),
            pltpu.VMEM((K, D), jnp.float32),
            pltpu.VMEM_SHARED((N_ACC, D), jnp.float32),
            pltpu.SemaphoreType.DMA,
        ],
    )
    def agg_kernel(h_hbm, src_hbm, dst_hbm, zeros_hbm, out_hbm,
                   src_v, dst_v, rows_v, acc, gsem):
        c = lax.axis_index("c")
        s = lax.axis_index("s")
        wid = c * NS + s
        pltpu.sync_copy(
            zeros_hbm.at[pl.ds(s * RPT, RPT)],
            acc.at[pl.ds(s * RPT, RPT)],
        )
        plsc.subcore_barrier()

        def body(a, carry):
            base = (wid * cpt + a) * K
            pltpu.sync_copy(src_hbm.at[pl.ds(base, K)], src_v)
            pltpu.sync_copy(dst_hbm.at[pl.ds(base, K)], dst_v)
            pltpu.async_copy(h_hbm.at[src_v], rows_v, gsem).wait()
            pltpu.sync_copy(rows_v, acc.at[dst_v], add=True)
            return carry

        lax.fori_loop(0, cpt, body, 0)
        plsc.subcore_barrier()
        pltpu.sync_copy(
            acc.at[pl.ds(s * RPT, RPT)],
            out_hbm.at[c, pl.ds(s * RPT, RPT)],
        )

    return agg_kernel


def _tc_first_body(h_ref, x_ref, w_ref, dinv_ref, hp_ref):
    ones_w = jnp.ones((NW, 1), jnp.float32)
    deg = 1.0 + lax.dot_general(
        h_ref[...], ones_w, (((0,), (0,)), ((), ())),
        preferred_element_type=jnp.float32)
    dinv = lax.rsqrt(deg)
    y = jnp.dot(x_ref[...], w_ref[...], preferred_element_type=jnp.float32)
    dinv_ref[...] = dinv
    hp_ref[...] = y * dinv


_tc_first = pl.pallas_call(
    _tc_first_body,
    out_shape=[
        jax.ShapeDtypeStruct((N, 1), jnp.float32),
        jax.ShapeDtypeStruct((N, D), jnp.float32),
    ],
)


def _tc_mid_body(p0_ref, p1_ref, hp_ref, dinv_ref, b_ref, g_ref, be_ref,
                 w_ref, out_ref):
    dinv = dinv_ref[...]
    t = dinv * (p0_ref[...] + p1_ref[...] + hp_ref[...]) + b_ref[...]
    m = jnp.mean(t, axis=0, keepdims=True)
    cen = t - m
    v = jnp.mean(cen * cen, axis=0, keepdims=True)
    tn = cen * lax.rsqrt(v + 1e-5) * g_ref[...] + be_ref[...]
    h = jnp.maximum(tn, 0.0)
    y = jnp.dot(h, w_ref[...], preferred_element_type=jnp.float32)
    out_ref[...] = y * dinv


_tc_mid = pl.pallas_call(
    _tc_mid_body,
    out_shape=jax.ShapeDtypeStruct((N, D), jnp.float32),
)


def _tc_final_body(p0_ref, p1_ref, hp_ref, dinv_ref, b_ref, out_ref):
    t = dinv_ref[...] * (p0_ref[...] + p1_ref[...] + hp_ref[...]) + b_ref[...]
    mx = jnp.max(t, axis=1, keepdims=True)
    ex = jnp.exp(t - mx)
    lse = jnp.log(jnp.sum(ex, axis=1, keepdims=True)) + mx
    out_ref[...] = t - lse


_tc_final = pl.pallas_call(
    _tc_final_body,
    out_shape=jax.ShapeDtypeStruct((N, D), jnp.float32),
)


def kernel(x, edge_index, W1, b1, g1, be1, W2, b2, g2, be2, Wl, bl):
    e = edge_index.shape[1]
    cpt = _cdiv(e, NW * K)
    e_pad = NW * K * cpt
    pad = e_pad - e

    src = edge_index[0].astype(jnp.int32)
    dst = edge_index[1].astype(jnp.int32)
    src_p = jnp.concatenate([src, jnp.zeros((pad,), jnp.int32)])
    dst_p = jnp.concatenate([dst, jnp.full((pad,), N, jnp.int32)])

    zeros_agg = jnp.zeros((N_ACC, D), jnp.float32)

    hists = _make_deg_kernel(cpt)(dst_p)[:, 0, :N]

    b1r = b1.reshape(1, D)
    g1r = g1.reshape(1, D)
    be1r = be1.reshape(1, D)
    b2r = b2.reshape(1, D)
    g2r = g2.reshape(1, D)
    be2r = be2.reshape(1, D)
    blr = bl.reshape(1, D)

    agg = _make_agg_kernel(cpt)

    dinv, hp1 = _tc_first(hists, x, W1)
    p1 = agg(hp1, src_p, dst_p, zeros_agg)
    hp2 = _tc_mid(p1[0, :N], p1[1, :N], hp1, dinv, b1r, g1r, be1r, W2)
    p2 = agg(hp2, src_p, dst_p, zeros_agg)
    hp3 = _tc_mid(p2[0, :N], p2[1, :N], hp2, dinv, b2r, g2r, be2r, Wl)
    p3 = agg(hp3, src_p, dst_p, zeros_agg)
    return _tc_final(p3[0, :N], p3[1, :N], hp3, dinv, blr)
